# baseline (speedup 1.0000x reference)
import functools
import math

import jax
import jax.numpy as jnp
from jax import lax
from jax.experimental import pallas as pl
from jax.experimental.pallas import tpu as pltpu
from jax.experimental.pallas import tpu_sc as plsc

D_MODEL = 768
N_HEADS = 12
HD = D_MODEL // N_HEADS
TL = 512
UPAD = 64
OUT_CHUNK = 1024

_f32 = jnp.float32


def _dot(a, b, ca, cb):
    return lax.dot_general(a, b, (((ca,), (cb,)), ((), ())),
                           preferred_element_type=_f32)


def _pass1_body(u, x_ref, xs_ref, wq_ref, bq_ref, wk_ref, bk_ref,
                m_ref, ks_scr):
    t = pl.program_id(1)

    @pl.when(t == 0)
    def _():
        ks_scr[...] = _dot(xs_ref[0], wk_ref[...], 1, 1) + bk_ref[...]

    q = _dot(x_ref[0], wq_ref[...], 1, 1) + bq_ref[...]
    row = lax.broadcasted_iota(jnp.int32, (UPAD, TL), 0)
    valid = row < u
    for h in range(N_HEADS):
        sl = slice(h * HD, (h + 1) * HD)
        st = _dot(ks_scr[:, sl], q[:, sl], 1, 1)
        smax = jnp.max(jnp.where(valid, st, -jnp.inf), axis=0)
        ssum = jnp.sum(jnp.where(valid, st, 0.0), axis=0)
        m_ref[0, h, :] = smax - ssum * (1.0 / u)


def _pass1(x, x_s, Wq, bq, Wk, bk, u):
    B, L, d = x.shape
    grid = (B, L // TL)
    return pl.pallas_call(
        functools.partial(_pass1_body, u),
        grid=grid,
        in_specs=[
            pl.BlockSpec((1, TL, d), lambda b, t: (b, t, 0)),
            pl.BlockSpec((1, UPAD, d), lambda b, t: (b, 0, 0)),
            pl.BlockSpec((d, d), lambda b, t: (0, 0)),
            pl.BlockSpec((d,), lambda b, t: (0,)),
            pl.BlockSpec((d, d), lambda b, t: (0, 0)),
            pl.BlockSpec((d,), lambda b, t: (0,)),
        ],
        out_specs=pl.BlockSpec((1, N_HEADS, TL), lambda b, t: (b, 0, t)),
        out_shape=jax.ShapeDtypeStruct((B, N_HEADS, L), _f32),
        scratch_shapes=[pltpu.VMEM((UPAD, d), _f32)],
    )(x, x_s, Wq, bq, Wk, bk)


def _topk_body(u, L, BH, m_ref, idx_ref, v_scr):
    v_scr[...] = m_ref[...]
    gidx = lax.broadcasted_iota(jnp.int32, (BH, L), 1)
    lane = lax.broadcasted_iota(jnp.int32, (BH, 128), 1)

    def body(j, orow):
        v = v_scr[...]
        mx = jnp.max(v, axis=1, keepdims=True)
        am = jnp.min(jnp.where(v == mx, gidx, jnp.int32(L)),
                     axis=1, keepdims=True)
        orow = jnp.where(lane == j, am, orow)
        v_scr[...] = jnp.where(gidx == am, -jnp.inf, v)
        return orow

    orow = lax.fori_loop(0, u, body, jnp.zeros((BH, 128), jnp.int32))
    idx_ref[...] = orow


def _topk(M, u):
    BH = M.shape[0] * M.shape[1]
    L = M.shape[2]
    return pl.pallas_call(
        functools.partial(_topk_body, u, L, BH),
        grid=(1,),
        in_specs=[pl.BlockSpec((BH, L), lambda i: (0, 0))],
        out_specs=pl.BlockSpec((BH, 128), lambda i: (0, 0)),
        out_shape=jax.ShapeDtypeStruct((BH, 128), jnp.int32),
        scratch_shapes=[pltpu.VMEM((BH, L), _f32)],
    )(M.reshape(BH, L))


def _gather_rows(xflat, gidx):
    info = plsc.get_sparse_core_info()
    nw = info.num_cores * info.num_subcores
    n, d = gidx.shape[0], xflat.shape[1]
    bpw = n // nw
    mesh = plsc.VectorSubcoreMesh(core_axis_name="c", subcore_axis_name="s")

    @functools.partial(
        pl.kernel, mesh=mesh,
        out_type=jax.ShapeDtypeStruct((n, d), _f32),
        scratch_types=[
            pltpu.VMEM((bpw,), jnp.int32),
            pltpu.VMEM((bpw, d), _f32),
            pltpu.SemaphoreType.DMA,
        ],
    )
    def k(x_hbm, idx_hbm, out_hbm, idx_v, rows_v, sem):
        wid = lax.axis_index("s") * info.num_cores + lax.axis_index("c")
        base = wid * bpw
        pltpu.sync_copy(idx_hbm.at[pl.ds(base, bpw)], idx_v)
        pltpu.async_copy(x_hbm.at[idx_v], rows_v, sem).wait()
        pltpu.sync_copy(rows_v, out_hbm.at[pl.ds(base, bpw)])

    return k(xflat, gidx)


def _pass2_body(scale, nt, x_ref, xsel_ref, wq_ref, bq_ref, wk_ref, bk_ref,
                wv_ref, bv_ref, ored_ref, vs_ref,
                qred_scr, m_scr, l_scr, acc_scr, vsum_scr):
    t = pl.program_id(1)
    R = N_HEADS * UPAD

    @pl.when(t == 0)
    def _():
        for h in range(N_HEADS):
            rs = slice(h * UPAD, (h + 1) * UPAD)
            cs = slice(h * HD, (h + 1) * HD)
            qred_scr[rs, :] = (_dot(xsel_ref[0, rs, :], wq_ref[cs, :], 1, 1)
                               + bq_ref[pl.ds(h * HD, HD)])
        m_scr[...] = jnp.full((R, 1), -1e30, _f32)
        l_scr[...] = jnp.zeros((R, 1), _f32)
        acc_scr[...] = jnp.zeros((R, HD), _f32)
        vsum_scr[...] = jnp.zeros((N_HEADS, HD), _f32)

    xt = x_ref[0].astype(jnp.bfloat16)
    kt = _dot(xt, wk_ref[...], 1, 1) + bk_ref[...]
    vt = _dot(xt, wv_ref[...], 1, 1) + bv_ref[...]
    for h in range(N_HEADS):
        rs = slice(h * UPAD, (h + 1) * UPAD)
        cs = slice(h * HD, (h + 1) * HD)
        kth = kt[:, cs]
        vth = vt[:, cs]
        s = _dot(qred_scr[rs, :], kth, 1, 1) * scale
        m_old = m_scr[rs, :]
        m_new = jnp.maximum(m_old, jnp.max(s, axis=1, keepdims=True))
        p = jnp.exp(s - m_new)
        corr = jnp.exp(m_old - m_new)
        l_scr[rs, :] = l_scr[rs, :] * corr + jnp.sum(p, axis=1, keepdims=True)
        acc_scr[rs, :] = acc_scr[rs, :] * corr + _dot(p, vth, 1, 0)
        m_scr[rs, :] = m_new
        vsum_scr[h:h + 1, :] = (vsum_scr[h:h + 1, :]
                                + jnp.sum(vth, axis=0, keepdims=True))

    @pl.when(t == nt - 1)
    def _():
        ored_ref[0] = acc_scr[...] / l_scr[...]
        vs_ref[0] = vsum_scr[...]


def _pass2(x, x_sel, Wq, bq, Wk, bk, Wv, bv):
    B, L, d = x.shape
    nt = L // TL
    scale = HD ** (-0.5)
    R = N_HEADS * UPAD
    return pl.pallas_call(
        functools.partial(_pass2_body, scale, nt),
        grid=(B, nt),
        in_specs=[
            pl.BlockSpec((1, TL, d), lambda b, t: (b, t, 0)),
            pl.BlockSpec((1, R, d), lambda b, t: (b, 0, 0)),
            pl.BlockSpec((d, d), lambda b, t: (0, 0)),
            pl.BlockSpec((d,), lambda b, t: (0,)),
            pl.BlockSpec((d, d), lambda b, t: (0, 0)),
            pl.BlockSpec((d,), lambda b, t: (0,)),
            pl.BlockSpec((d, d), lambda b, t: (0, 0)),
            pl.BlockSpec((d,), lambda b, t: (0,)),
        ],
        out_specs=[
            pl.BlockSpec((1, R, HD), lambda b, t: (b, 0, 0)),
            pl.BlockSpec((1, N_HEADS, HD), lambda b, t: (b, 0, 0)),
        ],
        out_shape=[
            jax.ShapeDtypeStruct((B, R, HD), _f32),
            jax.ShapeDtypeStruct((B, N_HEADS, HD), _f32),
        ],
        scratch_shapes=[
            pltpu.VMEM((R, HD), _f32),
            pltpu.VMEM((R, 1), _f32),
            pltpu.VMEM((R, 1), _f32),
            pltpu.VMEM((R, HD), _f32),
            pltpu.VMEM((N_HEADS, HD), _f32),
        ],
    )(x, x_sel, Wq, bq, Wk, bk, Wv, bv)


def _final_body(u, L, ored_ref, vs_ref, idx_ref, wo_ref, bo_ref, out_ref,
                d_scr):
    d = D_MODEL
    base = bo_ref[...][None, :]
    for h in range(N_HEADS):
        rs = slice(h * UPAD, (h + 1) * UPAD)
        cs = slice(h * HD, (h + 1) * HD)
        vmh = vs_ref[0, h:h + 1, :] * (1.0 / L)
        woh = wo_ref[:, cs]
        base = base + _dot(vmh, woh, 1, 1)
        d_scr[rs, :] = _dot(ored_ref[0, rs, :] - vmh, woh, 1, 1)

    out_ref[0] = jnp.broadcast_to(base, (L, d))

    def body(k, _):
        h = k // u
        j = k - h * u
        i = idx_ref[0, h, j]
        out_ref[0, pl.ds(i, 1), :] = (out_ref[0, pl.ds(i, 1), :]
                                      + d_scr[pl.ds(h * UPAD + j, 1), :])
        return 0

    lax.fori_loop(0, N_HEADS * u, body, 0)


def _final(out_red, vsum, idx_local, Wo, bo, u, L):
    B = out_red.shape[0]
    d = D_MODEL
    R = N_HEADS * UPAD
    return pl.pallas_call(
        functools.partial(_final_body, u, L),
        grid=(B,),
        in_specs=[
            pl.BlockSpec((1, R, HD), lambda b: (b, 0, 0)),
            pl.BlockSpec((1, N_HEADS, HD), lambda b: (b, 0, 0)),
            pl.BlockSpec((1, N_HEADS, UPAD), lambda b: (b, 0, 0),
                         memory_space=pltpu.SMEM),
            pl.BlockSpec((d, d), lambda b: (0, 0)),
            pl.BlockSpec((d,), lambda b: (0,)),
        ],
        out_specs=pl.BlockSpec((1, L, d), lambda b: (b, 0, 0)),
        out_shape=jax.ShapeDtypeStruct((B, L, d), _f32),
        scratch_shapes=[
            pltpu.VMEM((R, d), _f32),
        ],
    )(out_red, vsum, idx_local, Wo, bo)


def kernel(x, Wq, bq, Wk, bk, Wv, bv, Wo, bo):
    B, L, d = x.shape
    u = max(1, min(5 * int(math.ceil(math.log(max(L, 2)))), L))

    perm = jax.random.permutation(jax.random.key(42), L)[:u]
    x_s = jnp.zeros((B, UPAD, d), _f32).at[:, :u, :].set(x[:, perm, :])

    M = _pass1(x, x_s, Wq, bq, Wk, bk, u)

    idxp = _topk(M, u)
    local = idxp[:, :UPAD].reshape(B, N_HEADS, UPAD)
    gidx = (local + (jnp.arange(B, dtype=jnp.int32) * L)[:, None, None])
    x_sel = _gather_rows(x.reshape(B * L, d), gidx.reshape(-1))
    x_sel = x_sel.reshape(B, N_HEADS * UPAD, d)

    out_red, vsum = _pass2(x, x_sel, Wq, bq,
                           Wk.astype(jnp.bfloat16), bk,
                           Wv.astype(jnp.bfloat16), bv)
    return _final(out_red, vsum, local, Wo, bo, u, L)


# speedup vs baseline: 1.4135x; 1.4135x over previous
import functools
import math

import jax
import jax.numpy as jnp
from jax import lax
from jax.experimental import pallas as pl
from jax.experimental.pallas import tpu as pltpu
from jax.experimental.pallas import tpu_sc as plsc

D_MODEL = 768
N_HEADS = 12
HD = D_MODEL // N_HEADS
TL = 512
UPAD = 64
OUT_CHUNK = 1024

_f32 = jnp.float32


def _dot(a, b, ca, cb):
    return lax.dot_general(a, b, (((ca,), (cb,)), ((), ())),
                           preferred_element_type=_f32)


def _pass1_body(u, x_ref, xs_ref, wq_ref, bq_ref, wk_ref, bk_ref,
                m_ref, ks_scr):
    t = pl.program_id(1)

    @pl.when(t == 0)
    def _():
        ks_scr[...] = _dot(xs_ref[0], wk_ref[...], 1, 1) + bk_ref[...]

    q = _dot(x_ref[0], wq_ref[...], 1, 1) + bq_ref[...]
    row = lax.broadcasted_iota(jnp.int32, (UPAD, TL), 0)
    valid = row < u
    for h in range(N_HEADS):
        sl = slice(h * HD, (h + 1) * HD)
        st = _dot(ks_scr[:, sl], q[:, sl], 1, 1)
        smax = jnp.max(jnp.where(valid, st, -jnp.inf), axis=0)
        ssum = jnp.sum(jnp.where(valid, st, 0.0), axis=0)
        m_ref[0, h, :] = smax - ssum * (1.0 / u)


def _pass1(x, x_s, Wq, bq, Wk, bk, u):
    B, L, d = x.shape
    grid = (B, L // TL)
    return pl.pallas_call(
        functools.partial(_pass1_body, u),
        grid=grid,
        in_specs=[
            pl.BlockSpec((1, TL, d), lambda b, t: (b, t, 0)),
            pl.BlockSpec((1, UPAD, d), lambda b, t: (b, 0, 0)),
            pl.BlockSpec((d, d), lambda b, t: (0, 0)),
            pl.BlockSpec((d,), lambda b, t: (0,)),
            pl.BlockSpec((d, d), lambda b, t: (0, 0)),
            pl.BlockSpec((d,), lambda b, t: (0,)),
        ],
        out_specs=pl.BlockSpec((1, N_HEADS, TL), lambda b, t: (b, 0, t)),
        out_shape=jax.ShapeDtypeStruct((B, N_HEADS, L), _f32),
        scratch_shapes=[pltpu.VMEM((UPAD, d), _f32)],
    )(x, x_s, Wq, bq, Wk, bk)


def _topk_body(u, L, BH, m_ref, idx_ref, v_scr):
    v_scr[...] = m_ref[...]
    gidx = lax.broadcasted_iota(jnp.int32, (BH, L), 1)
    lane = lax.broadcasted_iota(jnp.int32, (BH, 128), 1)

    def body(j, orow):
        v = v_scr[...]
        mx = jnp.max(v, axis=1, keepdims=True)
        am = jnp.min(jnp.where(v == mx, gidx, jnp.int32(L)),
                     axis=1, keepdims=True)
        orow = jnp.where(lane == j, am, orow)
        v_scr[...] = jnp.where(gidx == am, -jnp.inf, v)
        return orow

    orow = lax.fori_loop(0, u, body, jnp.zeros((BH, 128), jnp.int32))
    idx_ref[...] = orow


def _topk(M, u):
    BH = M.shape[0] * M.shape[1]
    L = M.shape[2]
    return pl.pallas_call(
        functools.partial(_topk_body, u, L, BH),
        grid=(1,),
        in_specs=[pl.BlockSpec((BH, L), lambda i: (0, 0))],
        out_specs=pl.BlockSpec((BH, 128), lambda i: (0, 0)),
        out_shape=jax.ShapeDtypeStruct((BH, 128), jnp.int32),
        scratch_shapes=[pltpu.VMEM((BH, L), _f32)],
    )(M.reshape(BH, L))


def _gather_rows(xflat, gidx):
    info = plsc.get_sparse_core_info()
    nw = info.num_cores * info.num_subcores
    n, d = gidx.shape[0], xflat.shape[1]
    bpw = n // nw
    mesh = plsc.VectorSubcoreMesh(core_axis_name="c", subcore_axis_name="s")

    @functools.partial(
        pl.kernel, mesh=mesh,
        out_type=jax.ShapeDtypeStruct((n, d), _f32),
        scratch_types=[
            pltpu.VMEM((bpw,), jnp.int32),
            pltpu.VMEM((bpw, d), _f32),
            pltpu.SemaphoreType.DMA,
        ],
    )
    def k(x_hbm, idx_hbm, out_hbm, idx_v, rows_v, sem):
        wid = lax.axis_index("s") * info.num_cores + lax.axis_index("c")
        base = wid * bpw
        pltpu.sync_copy(idx_hbm.at[pl.ds(base, bpw)], idx_v)
        pltpu.async_copy(x_hbm.at[idx_v], rows_v, sem).wait()
        pltpu.sync_copy(rows_v, out_hbm.at[pl.ds(base, bpw)])

    return k(xflat, gidx)


TL2 = 1024


def _pass2_body(scale, nt, x_ref, xsel_ref, wq_ref, bq_ref, wk_ref, bk_ref,
                wv_ref, bv_ref, ored_ref, vs_ref,
                a_scr, c_scr, m_scr, l_scr, acc_scr, vsum_scr):
    t = pl.program_id(1)
    R = N_HEADS * UPAD

    @pl.when(t == 0)
    def _():
        for h in range(N_HEADS):
            rs = slice(h * UPAD, (h + 1) * UPAD)
            ws = slice(h * HD, (h + 1) * HD)
            qred_h = (_dot(xsel_ref[0, rs, :], wq_ref[ws, :], 1, 1)
                      + bq_ref[pl.ds(h * HD, HD)])
            wk_h = wk_ref[ws, :]
            a_h = _dot(qred_h, wk_h, 1, 0) * scale
            a_scr[rs, :] = a_h.astype(jnp.bfloat16)
            bk_h = jnp.reshape(bk_ref[pl.ds(h * HD, HD)], (HD, 1))
            c_scr[rs, :] = _dot(qred_h, bk_h, 1, 0) * scale
        m_scr[...] = jnp.full((R, 1), -1e30, _f32)
        l_scr[...] = jnp.zeros((R, 1), _f32)
        acc_scr[...] = jnp.zeros((R, HD), _f32)
        vsum_scr[...] = jnp.zeros((8, D_MODEL), _f32)

    xt = x_ref[0].astype(jnp.bfloat16)
    s = _dot(a_scr[...], xt, 1, 1) + c_scr[...]
    vt = _dot(xt, wv_ref[...], 1, 1) + bv_ref[...]
    m_old = m_scr[...]
    m_new = jnp.maximum(m_old, jnp.max(s, axis=1, keepdims=True))
    p = jnp.exp(s - m_new)
    corr = jnp.exp(m_old - m_new)
    l_scr[...] = l_scr[...] * corr + jnp.sum(p, axis=1, keepdims=True)
    m_scr[...] = m_new
    for h in range(N_HEADS):
        rs = slice(h * UPAD, (h + 1) * UPAD)
        cs = slice(h * HD, (h + 1) * HD)
        acc_scr[rs, :] = (acc_scr[rs, :] * corr[rs, :]
                          + _dot(p[rs, :], vt[:, cs], 1, 0))
    vsum_scr[0:1, :] = vsum_scr[0:1, :] + jnp.sum(vt, axis=0, keepdims=True)

    @pl.when(t == nt - 1)
    def _():
        ored_ref[0] = acc_scr[...] / l_scr[...]
        vs_ref[0] = jnp.broadcast_to(vsum_scr[0:1, :], (8, D_MODEL))


def _pass2(x, x_sel, Wq, bq, Wk, bk, Wv, bv):
    B, L, d = x.shape
    nt = L // TL2
    scale = HD ** (-0.5)
    R = N_HEADS * UPAD
    return pl.pallas_call(
        functools.partial(_pass2_body, scale, nt),
        grid=(B, nt),
        in_specs=[
            pl.BlockSpec((1, TL2, d), lambda b, t: (b, t, 0)),
            pl.BlockSpec((1, R, d), lambda b, t: (b, 0, 0)),
            pl.BlockSpec((d, d), lambda b, t: (0, 0)),
            pl.BlockSpec((d,), lambda b, t: (0,)),
            pl.BlockSpec((d, d), lambda b, t: (0, 0)),
            pl.BlockSpec((d,), lambda b, t: (0,)),
            pl.BlockSpec((d, d), lambda b, t: (0, 0)),
            pl.BlockSpec((d,), lambda b, t: (0,)),
        ],
        out_specs=[
            pl.BlockSpec((1, R, HD), lambda b, t: (b, 0, 0)),
            pl.BlockSpec((1, 8, d), lambda b, t: (b, 0, 0)),
        ],
        out_shape=[
            jax.ShapeDtypeStruct((B, R, HD), _f32),
            jax.ShapeDtypeStruct((B, 8, d), _f32),
        ],
        scratch_shapes=[
            pltpu.VMEM((R, d), jnp.bfloat16),
            pltpu.VMEM((R, 1), _f32),
            pltpu.VMEM((R, 1), _f32),
            pltpu.VMEM((R, 1), _f32),
            pltpu.VMEM((R, HD), _f32),
            pltpu.VMEM((8, d), _f32),
        ],
    )(x, x_sel, Wq, bq, Wk, bk, Wv, bv)


def _final_body(u, L, ored_ref, vs_ref, idx_ref, wo_ref, bo_ref, out_ref,
                d_scr):
    d = D_MODEL
    base = bo_ref[...][None, :]
    for h in range(N_HEADS):
        rs = slice(h * UPAD, (h + 1) * UPAD)
        cs = slice(h * HD, (h + 1) * HD)
        vmh = vs_ref[0, h:h + 1, :] * (1.0 / L)
        woh = wo_ref[:, cs]
        base = base + _dot(vmh, woh, 1, 1)
        d_scr[rs, :] = _dot(ored_ref[0, rs, :] - vmh, woh, 1, 1)

    out_ref[0] = jnp.broadcast_to(base, (L, d))

    def body(k, _):
        h = k // u
        j = k - h * u
        i = idx_ref[0, h, j]
        out_ref[0, pl.ds(i, 1), :] = (out_ref[0, pl.ds(i, 1), :]
                                      + d_scr[pl.ds(h * UPAD + j, 1), :])
        return 0

    lax.fori_loop(0, N_HEADS * u, body, 0)


def _final(out_red, vsum, idx_local, Wo, bo, u, L):
    B = out_red.shape[0]
    d = D_MODEL
    R = N_HEADS * UPAD
    return pl.pallas_call(
        functools.partial(_final_body, u, L),
        grid=(B,),
        in_specs=[
            pl.BlockSpec((1, R, HD), lambda b: (b, 0, 0)),
            pl.BlockSpec((1, N_HEADS, HD), lambda b: (b, 0, 0)),
            pl.BlockSpec((1, N_HEADS, UPAD), lambda b: (b, 0, 0),
                         memory_space=pltpu.SMEM),
            pl.BlockSpec((d, d), lambda b: (0, 0)),
            pl.BlockSpec((d,), lambda b: (0,)),
        ],
        out_specs=pl.BlockSpec((1, L, d), lambda b: (b, 0, 0)),
        out_shape=jax.ShapeDtypeStruct((B, L, d), _f32),
        scratch_shapes=[
            pltpu.VMEM((R, d), _f32),
        ],
    )(out_red, vsum, idx_local, Wo, bo)


def kernel(x, Wq, bq, Wk, bk, Wv, bv, Wo, bo):
    B, L, d = x.shape
    u = max(1, min(5 * int(math.ceil(math.log(max(L, 2)))), L))

    perm = jax.random.permutation(jax.random.key(42), L)[:u]
    x_s = jnp.zeros((B, UPAD, d), _f32).at[:, :u, :].set(x[:, perm, :])

    M = _pass1(x, x_s, Wq, bq, Wk, bk, u)

    idxp = _topk(M, u)
    local = idxp[:, :UPAD].reshape(B, N_HEADS, UPAD)
    gidx = (local + (jnp.arange(B, dtype=jnp.int32) * L)[:, None, None])
    x_sel = _gather_rows(x.reshape(B * L, d), gidx.reshape(-1))
    x_sel = x_sel.reshape(B, N_HEADS * UPAD, d)

    out_red, vsum8 = _pass2(x, x_sel, Wq, bq, Wk, bk,
                            Wv.astype(jnp.bfloat16), bv)
    vsum = vsum8[:, 0, :].reshape(B, N_HEADS, HD)
    return _final(out_red, vsum, local, Wo, bo, u, L)


# speedup vs baseline: 1.5167x; 1.0730x over previous
import functools
import math

import jax
import jax.numpy as jnp
from jax import lax
from jax.experimental import pallas as pl
from jax.experimental.pallas import tpu as pltpu
from jax.experimental.pallas import tpu_sc as plsc

D_MODEL = 768
N_HEADS = 12
HD = D_MODEL // N_HEADS
TL = 1024
UPAD = 64
OUT_CHUNK = 1024

_f32 = jnp.float32


def _dot(a, b, ca, cb):
    return lax.dot_general(a, b, (((ca,), (cb,)), ((), ())),
                           preferred_element_type=_f32)


def _pass1_body(u, x_ref, xs_ref, wq_ref, bq_ref, wk_ref, bk_ref,
                m_ref, w1_scr, c1_scr):
    t = pl.program_id(1)
    R = N_HEADS * UPAD

    @pl.when(t == 0)
    def _():
        ks = _dot(xs_ref[0], wk_ref[...], 1, 1) + bk_ref[...]
        for h in range(N_HEADS):
            rs = slice(h * UPAD, (h + 1) * UPAD)
            ws = slice(h * HD, (h + 1) * HD)
            ks_h = ks[:, ws]
            w1_scr[rs, :] = _dot(ks_h, wq_ref[ws, :], 1, 0)
            bq_h = jnp.reshape(bq_ref[pl.ds(h * HD, HD)], (HD, 1))
            c1_scr[rs, :] = _dot(ks_h, bq_h, 1, 0)

    st_all = _dot(w1_scr[...], x_ref[0], 1, 1) + c1_scr[...]
    row = lax.broadcasted_iota(jnp.int32, (UPAD, TL), 0)
    valid = row < u
    for h in range(N_HEADS):
        rs = slice(h * UPAD, (h + 1) * UPAD)
        st = st_all[rs, :]
        smax = jnp.max(jnp.where(valid, st, -jnp.inf), axis=0)
        ssum = jnp.sum(jnp.where(valid, st, 0.0), axis=0)
        m_ref[0, h, :] = smax - ssum * (1.0 / u)


def _pass1(x, x_s, Wq, bq, Wk, bk, u):
    B, L, d = x.shape
    R = N_HEADS * UPAD
    grid = (B, L // TL)
    return pl.pallas_call(
        functools.partial(_pass1_body, u),
        grid=grid,
        in_specs=[
            pl.BlockSpec((1, TL, d), lambda b, t: (b, t, 0)),
            pl.BlockSpec((1, UPAD, d), lambda b, t: (b, 0, 0)),
            pl.BlockSpec((d, d), lambda b, t: (0, 0)),
            pl.BlockSpec((d,), lambda b, t: (0,)),
            pl.BlockSpec((d, d), lambda b, t: (0, 0)),
            pl.BlockSpec((d,), lambda b, t: (0,)),
        ],
        out_specs=pl.BlockSpec((1, N_HEADS, TL), lambda b, t: (b, 0, t)),
        out_shape=jax.ShapeDtypeStruct((B, N_HEADS, L), _f32),
        scratch_shapes=[pltpu.VMEM((R, d), _f32),
                        pltpu.VMEM((R, 1), _f32)],
    )(x, x_s, Wq, bq, Wk, bk)


def _topk_body(u, L, BH, m_ref, idx_ref, v_scr):
    v_scr[...] = m_ref[...]
    gidx = lax.broadcasted_iota(jnp.int32, (BH, L), 1)
    lane = lax.broadcasted_iota(jnp.int32, (BH, 128), 1)

    def body(j, orow):
        v = v_scr[...]
        mx = jnp.max(v, axis=1, keepdims=True)
        am = jnp.min(jnp.where(v == mx, gidx, jnp.int32(L)),
                     axis=1, keepdims=True)
        orow = jnp.where(lane == j, am, orow)
        v_scr[...] = jnp.where(gidx == am, -jnp.inf, v)
        return orow

    orow = lax.fori_loop(0, u, body, jnp.zeros((BH, 128), jnp.int32))
    idx_ref[...] = orow


def _topk(M, u):
    BH = M.shape[0] * M.shape[1]
    L = M.shape[2]
    return pl.pallas_call(
        functools.partial(_topk_body, u, L, BH),
        grid=(1,),
        in_specs=[pl.BlockSpec((BH, L), lambda i: (0, 0))],
        out_specs=pl.BlockSpec((BH, 128), lambda i: (0, 0)),
        out_shape=jax.ShapeDtypeStruct((BH, 128), jnp.int32),
        scratch_shapes=[pltpu.VMEM((BH, L), _f32)],
    )(M.reshape(BH, L))


def _gather_rows(xflat, gidx):
    info = plsc.get_sparse_core_info()
    nw = info.num_cores * info.num_subcores
    n, d = gidx.shape[0], xflat.shape[1]
    bpw = n // nw
    mesh = plsc.VectorSubcoreMesh(core_axis_name="c", subcore_axis_name="s")

    @functools.partial(
        pl.kernel, mesh=mesh,
        out_type=jax.ShapeDtypeStruct((n, d), _f32),
        scratch_types=[
            pltpu.VMEM((bpw,), jnp.int32),
            pltpu.VMEM((bpw, d), _f32),
            pltpu.SemaphoreType.DMA,
        ],
    )
    def k(x_hbm, idx_hbm, out_hbm, idx_v, rows_v, sem):
        wid = lax.axis_index("s") * info.num_cores + lax.axis_index("c")
        base = wid * bpw
        pltpu.sync_copy(idx_hbm.at[pl.ds(base, bpw)], idx_v)
        pltpu.async_copy(x_hbm.at[idx_v], rows_v, sem).wait()
        pltpu.sync_copy(rows_v, out_hbm.at[pl.ds(base, bpw)])

    return k(xflat, gidx)


TL2 = 1024


def _pass2_body(scale, nt, x_ref, xsel_ref, wq_ref, bq_ref, wk_ref, bk_ref,
                wv_ref, bv_ref, ored_ref, vs_ref,
                a_scr, c_scr, wv_scr, m_scr, l_scr, acc_scr, vsum_scr):
    t = pl.program_id(1)
    R = N_HEADS * UPAD

    @pl.when(t == 0)
    def _():
        for h in range(N_HEADS):
            rs = slice(h * UPAD, (h + 1) * UPAD)
            ws = slice(h * HD, (h + 1) * HD)
            qred_h = (_dot(xsel_ref[0, rs, :], wq_ref[ws, :], 1, 1)
                      + bq_ref[pl.ds(h * HD, HD)])
            wk_h = wk_ref[ws, :]
            a_h = _dot(qred_h, wk_h, 1, 0) * scale
            a_scr[rs, :] = a_h.astype(jnp.bfloat16)
            bk_h = jnp.reshape(bk_ref[pl.ds(h * HD, HD)], (HD, 1))
            c_scr[rs, :] = _dot(qred_h, bk_h, 1, 0) * scale
        wv_scr[...] = wv_ref[...].astype(jnp.bfloat16)
        m_scr[...] = jnp.full((R, 1), -1e30, _f32)
        l_scr[...] = jnp.zeros((R, 1), _f32)
        acc_scr[...] = jnp.zeros((R, HD), _f32)
        vsum_scr[...] = jnp.zeros((8, D_MODEL), _f32)

    xt = x_ref[0].astype(jnp.bfloat16)
    s = _dot(a_scr[...], xt, 1, 1) + c_scr[...]
    vt = _dot(xt, wv_scr[...], 1, 1) + bv_ref[...]
    m_old = m_scr[...]
    m_new = jnp.maximum(m_old, jnp.max(s, axis=1, keepdims=True))
    p = jnp.exp(s - m_new)
    corr = jnp.exp(m_old - m_new)
    l_scr[...] = l_scr[...] * corr + jnp.sum(p, axis=1, keepdims=True)
    m_scr[...] = m_new
    for h in range(N_HEADS):
        rs = slice(h * UPAD, (h + 1) * UPAD)
        cs = slice(h * HD, (h + 1) * HD)
        acc_scr[rs, :] = (acc_scr[rs, :] * corr[rs, :]
                          + _dot(p[rs, :], vt[:, cs], 1, 0))
    vsum_scr[0:1, :] = vsum_scr[0:1, :] + jnp.sum(vt, axis=0, keepdims=True)

    @pl.when(t == nt - 1)
    def _():
        ored_ref[0] = acc_scr[...] / l_scr[...]
        vs_ref[0] = jnp.broadcast_to(vsum_scr[0:1, :], (8, D_MODEL))


def _pass2(x, x_sel, Wq, bq, Wk, bk, Wv, bv):
    B, L, d = x.shape
    nt = L // TL2
    scale = HD ** (-0.5)
    R = N_HEADS * UPAD
    return pl.pallas_call(
        functools.partial(_pass2_body, scale, nt),
        grid=(B, nt),
        in_specs=[
            pl.BlockSpec((1, TL2, d), lambda b, t: (b, t, 0)),
            pl.BlockSpec((1, R, d), lambda b, t: (b, 0, 0)),
            pl.BlockSpec((d, d), lambda b, t: (0, 0)),
            pl.BlockSpec((d,), lambda b, t: (0,)),
            pl.BlockSpec((d, d), lambda b, t: (0, 0)),
            pl.BlockSpec((d,), lambda b, t: (0,)),
            pl.BlockSpec((d, d), lambda b, t: (0, 0)),
            pl.BlockSpec((d,), lambda b, t: (0,)),
        ],
        out_specs=[
            pl.BlockSpec((1, R, HD), lambda b, t: (b, 0, 0)),
            pl.BlockSpec((1, 8, d), lambda b, t: (b, 0, 0)),
        ],
        out_shape=[
            jax.ShapeDtypeStruct((B, R, HD), _f32),
            jax.ShapeDtypeStruct((B, 8, d), _f32),
        ],
        scratch_shapes=[
            pltpu.VMEM((R, d), jnp.bfloat16),
            pltpu.VMEM((R, 1), _f32),
            pltpu.VMEM((d, d), jnp.bfloat16),
            pltpu.VMEM((R, 1), _f32),
            pltpu.VMEM((R, 1), _f32),
            pltpu.VMEM((R, HD), _f32),
            pltpu.VMEM((8, d), _f32),
        ],
    )(x, x_sel, Wq, bq, Wk, bk, Wv, bv)


def _final_body(u, L, ored_ref, vs_ref, idx_ref, wo_ref, bo_ref, out_ref,
                d_scr):
    d = D_MODEL
    base = bo_ref[...][None, :]
    for h in range(N_HEADS):
        rs = slice(h * UPAD, (h + 1) * UPAD)
        cs = slice(h * HD, (h + 1) * HD)
        vmh = vs_ref[0, h:h + 1, :] * (1.0 / L)
        woh = wo_ref[:, cs]
        base = base + _dot(vmh, woh, 1, 1)
        d_scr[rs, :] = _dot(ored_ref[0, rs, :] - vmh, woh, 1, 1)

    out_ref[0] = jnp.broadcast_to(base, (L, d))

    def body(k, _):
        h = k // u
        j = k - h * u
        i = idx_ref[0, h, j]
        out_ref[0, pl.ds(i, 1), :] = (out_ref[0, pl.ds(i, 1), :]
                                      + d_scr[pl.ds(h * UPAD + j, 1), :])
        return 0

    lax.fori_loop(0, N_HEADS * u, body, 0)


def _final(out_red, vsum, idx_local, Wo, bo, u, L):
    B = out_red.shape[0]
    d = D_MODEL
    R = N_HEADS * UPAD
    return pl.pallas_call(
        functools.partial(_final_body, u, L),
        grid=(B,),
        in_specs=[
            pl.BlockSpec((1, R, HD), lambda b: (b, 0, 0)),
            pl.BlockSpec((1, N_HEADS, HD), lambda b: (b, 0, 0)),
            pl.BlockSpec((1, N_HEADS, UPAD), lambda b: (b, 0, 0),
                         memory_space=pltpu.SMEM),
            pl.BlockSpec((d, d), lambda b: (0, 0)),
            pl.BlockSpec((d,), lambda b: (0,)),
        ],
        out_specs=pl.BlockSpec((1, L, d), lambda b: (b, 0, 0)),
        out_shape=jax.ShapeDtypeStruct((B, L, d), _f32),
        scratch_shapes=[
            pltpu.VMEM((R, d), _f32),
        ],
    )(out_red, vsum, idx_local, Wo, bo)


def kernel(x, Wq, bq, Wk, bk, Wv, bv, Wo, bo):
    B, L, d = x.shape
    u = max(1, min(5 * int(math.ceil(math.log(max(L, 2)))), L))

    perm = jax.random.permutation(jax.random.key(42), L)[:u]
    x_s = jnp.zeros((B, UPAD, d), _f32).at[:, :u, :].set(x[:, perm, :])

    M = _pass1(x, x_s, Wq, bq, Wk, bk, u)

    idxp = _topk(M, u)
    local = idxp[:, :UPAD].reshape(B, N_HEADS, UPAD)
    gidx = (local + (jnp.arange(B, dtype=jnp.int32) * L)[:, None, None])
    x_sel = _gather_rows(x.reshape(B * L, d), gidx.reshape(-1))
    x_sel = x_sel.reshape(B, N_HEADS * UPAD, d)

    out_red, vsum8 = _pass2(x, x_sel, Wq, bq, Wk, bk, Wv, bv)
    vsum = vsum8[:, 0, :].reshape(B, N_HEADS, HD)
    return _final(out_red, vsum, local, Wo, bo, u, L)
